# baseline (speedup 1.0000x reference)
import functools

import jax
import jax.numpy as jnp
from jax import lax
from jax.experimental import pallas as pl
from jax.experimental.pallas import tpu as pltpu
from jax.experimental.pallas import tpu_sc as plsc

B, L, D = 4096, 50, 32
N = B * L
NC, NS = 2, 16
NW = NC * NS
ROWS_PER_W = N // NW
CHUNK = 128
NCHUNK = ROWS_PER_W // CHUNK
NBUF = 5

_mesh = plsc.VectorSubcoreMesh(
    core_axis_name="c", subcore_axis_name="s", num_cores=NC, num_subcores=NS
)


@functools.partial(
    pl.kernel,
    out_type=jax.ShapeDtypeStruct((N, D), jnp.float32),
    mesh=_mesh,
    scratch_types=(
        [pltpu.VMEM((ROWS_PER_W,), jnp.int32)]
        + [pltpu.VMEM((CHUNK, D), jnp.float32) for _ in range(NBUF)]
        + [pltpu.SemaphoreType.DMA for _ in range(2 * NBUF)]
    ),
    compiler_params=pltpu.CompilerParams(use_tc_tiling_on_sc=False),
)
def _sc_gather(idx_hbm, table_hbm, out_hbm, idx_v, *bufs_and_sems):
    rows = bufs_and_sems[:NBUF]
    gsems = bufs_and_sems[NBUF : 2 * NBUF]
    osems = bufs_and_sems[2 * NBUF :]

    wid = lax.axis_index("s") * NC + lax.axis_index("c")
    chunk0 = wid * NCHUNK

    pltpu.sync_copy(idx_hbm.at[pl.ds(wid * ROWS_PER_W, ROWS_PER_W)], idx_v)

    def start_gather(j, b):
        idx_slice = idx_v.at[pl.ds(j * CHUNK, CHUNK)]
        pltpu.async_copy(table_hbm.at[idx_slice], rows[b], gsems[b])

    for b in range(NBUF):
        start_gather(b, b)

    @pl.loop(0, NCHUNK, step=NBUF)
    def _(g):
        for b in range(NBUF):
            j = g + b
            idx_slice = idx_v.at[pl.ds(j * CHUNK, CHUNK)]
            pltpu.make_async_copy(
                table_hbm.at[idx_slice], rows[b], gsems[b]
            ).wait()
            row0 = (chunk0 + j) * CHUNK
            out_slice = out_hbm.at[pl.ds(row0, CHUNK)]
            pltpu.async_copy(rows[b], out_slice, osems[b])
            pltpu.make_async_copy(rows[b], out_slice, osems[b]).wait()

            @pl.when(j + NBUF < NCHUNK)
            def _():
                start_gather(j + NBUF, b)


def kernel(word_inputs, feature_inputs, word_seq_lengths, char_inputs,
           char_seq_lengths, char_seq_recover, word_table,
           feat_table_0, feat_table_1):
    idx = word_inputs.reshape(-1).astype(jnp.int32)
    out = _sc_gather(idx, word_table)
    return out.reshape(B, L, D)


# speedup vs baseline: 1.0008x; 1.0008x over previous
import functools

import jax
import jax.numpy as jnp
from jax import lax
from jax.experimental import pallas as pl
from jax.experimental.pallas import tpu as pltpu
from jax.experimental.pallas import tpu_sc as plsc

B, L, D = 4096, 50, 32
N = B * L
NC, NS = 2, 16
NW = NC * NS
ROWS_PER_W = N // NW
CHUNK = 800
NCHUNK = ROWS_PER_W // CHUNK
NBUF = 4

_mesh = plsc.VectorSubcoreMesh(
    core_axis_name="c", subcore_axis_name="s", num_cores=NC, num_subcores=NS
)


@functools.partial(
    pl.kernel,
    out_type=jax.ShapeDtypeStruct((N, D), jnp.float32),
    mesh=_mesh,
    scratch_types=(
        [pltpu.VMEM((ROWS_PER_W,), jnp.int32)]
        + [pltpu.VMEM((CHUNK, D), jnp.float32) for _ in range(NBUF)]
        + [pltpu.SemaphoreType.DMA for _ in range(2 * NBUF)]
    ),
    compiler_params=pltpu.CompilerParams(use_tc_tiling_on_sc=False),
)
def _sc_gather(idx_hbm, table_hbm, out_hbm, idx_v, *bufs_and_sems):
    rows = bufs_and_sems[:NBUF]
    gsems = bufs_and_sems[NBUF : 2 * NBUF]
    osems = bufs_and_sems[2 * NBUF :]

    wid = lax.axis_index("s") * NC + lax.axis_index("c")
    chunk0 = wid * NCHUNK

    pltpu.sync_copy(idx_hbm.at[pl.ds(wid * ROWS_PER_W, ROWS_PER_W)], idx_v)

    def start_gather(j, b):
        idx_slice = idx_v.at[pl.ds(j * CHUNK, CHUNK)]
        pltpu.async_copy(table_hbm.at[idx_slice], rows[b], gsems[b])

    for b in range(NBUF):
        start_gather(b, b)

    @pl.loop(0, NCHUNK, step=NBUF)
    def _(g):
        for b in range(NBUF):
            j = g + b
            idx_slice = idx_v.at[pl.ds(j * CHUNK, CHUNK)]
            pltpu.make_async_copy(
                table_hbm.at[idx_slice], rows[b], gsems[b]
            ).wait()
            row0 = (chunk0 + j) * CHUNK
            out_slice = out_hbm.at[pl.ds(row0, CHUNK)]
            pltpu.async_copy(rows[b], out_slice, osems[b])
            pltpu.make_async_copy(rows[b], out_slice, osems[b]).wait()

            @pl.when(j + NBUF < NCHUNK)
            def _():
                start_gather(j + NBUF, b)


def kernel(word_inputs, feature_inputs, word_seq_lengths, char_inputs,
           char_seq_lengths, char_seq_recover, word_table,
           feat_table_0, feat_table_1):
    idx = word_inputs.reshape(-1).astype(jnp.int32)
    out = _sc_gather(idx, word_table)
    return out.reshape(B, L, D)
